# Initial kernel scaffold; baseline (speedup 1.0000x reference)
import jax
import jax.numpy as jnp
from jax.experimental import pallas as pl


def kernel(vecs, codebook):
    raise NotImplementedError("write your pallas kernel here")


# speedup vs baseline: 1.6561x; 1.6561x over previous
import functools

import jax
import jax.numpy as jnp
from jax.experimental import pallas as pl
from jax.experimental.pallas import tpu as pltpu

BATCH = 1024
EMBED = 768
PARTITION = 96
CENTROIDS = 256
DSUB = EMBED // PARTITION
PGROUP = 16


def _pq_body(v_ref, cb_ref, out_ref):
    outs = []
    for g in range(PGROUP):
        v_p = v_ref[:, g * DSUB:(g + 1) * DSUB]
        c_p = cb_ref[g]
        s = jax.lax.dot_general(
            v_p, c_p, (((1,), (1,)), ((), ())),
            precision=jax.lax.Precision.HIGHEST,
            preferred_element_type=jnp.float32)
        csq = 0.5 * jnp.sum(c_p * c_p, axis=-1)
        s = s - csq[None, :]
        m = jnp.max(s, axis=1, keepdims=True)
        iota = jax.lax.broadcasted_iota(jnp.int32, (BATCH, CENTROIDS), 1)
        idx = jnp.min(jnp.where(s == m, iota, CENTROIDS), axis=1,
                      keepdims=True)
        oh = (iota == idx).astype(jnp.float32)
        outs.append(jax.lax.dot_general(
            oh, c_p, (((1,), (0,)), ((), ())),
            precision=jax.lax.Precision.HIGHEST,
            preferred_element_type=jnp.float32))
    out_ref[...] = jnp.concatenate(outs, axis=1)


@functools.partial(jax.jit, static_argnames=())
def kernel(vecs, codebook):
    ngrp = PARTITION // PGROUP
    return pl.pallas_call(
        _pq_body,
        grid=(ngrp,),
        in_specs=[
            pl.BlockSpec((BATCH, PGROUP * DSUB), lambda i: (0, i)),
            pl.BlockSpec((PGROUP, CENTROIDS, DSUB), lambda i: (i, 0, 0)),
        ],
        out_specs=pl.BlockSpec((BATCH, PGROUP * DSUB), lambda i: (0, i)),
        out_shape=jax.ShapeDtypeStruct((BATCH, EMBED), jnp.float32),
    )(vecs, codebook)


# speedup vs baseline: 3.4622x; 2.0906x over previous
import functools

import jax
import jax.numpy as jnp
from jax.experimental import pallas as pl
from jax.experimental.pallas import tpu as pltpu

BATCH = 1024
EMBED = 768
PARTITION = 96
CENTROIDS = 256
DSUB = EMBED // PARTITION
PGROUP = 16


def _pq_body(v_ref, cb_ref, out_ref, cbd_ref, oh_ref):
    step = pl.program_id(0)

    @pl.when(step == 0)
    def _zero():
        cbd_ref[...] = jnp.zeros_like(cbd_ref)

    for g in range(PGROUP):
        cbd_ref[g * CENTROIDS:(g + 1) * CENTROIDS, g * DSUB:(g + 1) * DSUB] = cb_ref[g]

    cbd = cbd_ref[...]
    s = jax.lax.dot_general(
        v_ref[...], cbd, (((1,), (1,)), ((), ())),
        precision=jax.lax.Precision.HIGHEST,
        preferred_element_type=jnp.float32)

    iota = jax.lax.broadcasted_iota(jnp.int32, (BATCH, CENTROIDS), 1)
    for g in range(PGROUP):
        c_p = cb_ref[g]
        csq = 0.5 * jnp.sum(c_p * c_p, axis=-1)
        sg = s[:, g * CENTROIDS:(g + 1) * CENTROIDS] - csq[None, :]
        m = jnp.max(sg, axis=1, keepdims=True)
        idx = jnp.min(jnp.where(sg == m, iota, CENTROIDS), axis=1,
                      keepdims=True)
        oh_ref[:, g * CENTROIDS:(g + 1) * CENTROIDS] = (iota == idx).astype(jnp.float32)

    out_ref[...] = jax.lax.dot_general(
        oh_ref[...], cbd, (((1,), (0,)), ((), ())),
        precision=jax.lax.Precision.HIGHEST,
        preferred_element_type=jnp.float32)


@jax.jit
def kernel(vecs, codebook):
    ngrp = PARTITION // PGROUP
    return pl.pallas_call(
        _pq_body,
        grid=(ngrp,),
        in_specs=[
            pl.BlockSpec((BATCH, PGROUP * DSUB), lambda i: (0, i)),
            pl.BlockSpec((PGROUP, CENTROIDS, DSUB), lambda i: (i, 0, 0)),
        ],
        out_specs=pl.BlockSpec((BATCH, PGROUP * DSUB), lambda i: (0, i)),
        out_shape=jax.ShapeDtypeStruct((BATCH, EMBED), jnp.float32),
        scratch_shapes=[
            pltpu.VMEM((PGROUP * CENTROIDS, PGROUP * DSUB), jnp.float32),
            pltpu.VMEM((BATCH, PGROUP * CENTROIDS), jnp.float32),
        ],
    )(vecs, codebook)


# speedup vs baseline: 5.0181x; 1.4494x over previous
import jax
import jax.numpy as jnp
from jax.experimental import pallas as pl
from jax.experimental.pallas import tpu as pltpu

BATCH = 1024
EMBED = 768
PARTITION = 96
CENTROIDS = 256
DSUB = EMBED // PARTITION
PGROUP = 16

_CONTRACT_T = (((1,), (1,)), ((), ()))
_CONTRACT = (((1,), (0,)), ((), ()))


def _dot(a, b, dn):
    return jax.lax.dot_general(a, b, dn, preferred_element_type=jnp.float32)


def _pq_body(v_ref, cb_ref, out_ref, cbdh_ref, cbdl_ref, oh_ref):
    step = pl.program_id(0)

    @pl.when(step == 0)
    def _zero():
        cbdh_ref[...] = jnp.zeros_like(cbdh_ref)
        cbdl_ref[...] = jnp.zeros_like(cbdl_ref)

    for g in range(PGROUP):
        slab = cb_ref[g]
        hi = slab.astype(jnp.bfloat16)
        lo = (slab - hi.astype(jnp.float32)).astype(jnp.bfloat16)
        rows = slice(g * CENTROIDS, (g + 1) * CENTROIDS)
        cols = slice(g * DSUB, (g + 1) * DSUB)
        cbdh_ref[rows, cols] = hi
        cbdl_ref[rows, cols] = lo

    v = v_ref[...]
    vh = v.astype(jnp.bfloat16)
    vl = (v - vh.astype(jnp.float32)).astype(jnp.bfloat16)
    cbdh = cbdh_ref[...]
    cbdl = cbdl_ref[...]

    s = (_dot(vh, cbdh, _CONTRACT_T) + _dot(vh, cbdl, _CONTRACT_T)
         + _dot(vl, cbdh, _CONTRACT_T))

    iota = jax.lax.broadcasted_iota(jnp.int32, (BATCH, CENTROIDS), 1)
    for g in range(PGROUP):
        c_p = cb_ref[g]
        csq = 0.5 * jnp.sum(c_p * c_p, axis=-1)
        sg = s[:, g * CENTROIDS:(g + 1) * CENTROIDS] - csq[None, :]
        m = jnp.max(sg, axis=1, keepdims=True)
        idx = jnp.min(jnp.where(sg == m, iota, CENTROIDS), axis=1,
                      keepdims=True)
        oh_ref[:, g * CENTROIDS:(g + 1) * CENTROIDS] = (iota == idx).astype(jnp.bfloat16)

    oh = oh_ref[...]
    out_ref[...] = _dot(oh, cbdh, _CONTRACT) + _dot(oh, cbdl, _CONTRACT)


@jax.jit
def kernel(vecs, codebook):
    ngrp = PARTITION // PGROUP
    return pl.pallas_call(
        _pq_body,
        grid=(ngrp,),
        in_specs=[
            pl.BlockSpec((BATCH, PGROUP * DSUB), lambda i: (0, i)),
            pl.BlockSpec((PGROUP, CENTROIDS, DSUB), lambda i: (i, 0, 0)),
        ],
        out_specs=pl.BlockSpec((BATCH, PGROUP * DSUB), lambda i: (0, i)),
        out_shape=jax.ShapeDtypeStruct((BATCH, EMBED), jnp.float32),
        scratch_shapes=[
            pltpu.VMEM((PGROUP * CENTROIDS, PGROUP * DSUB), jnp.bfloat16),
            pltpu.VMEM((PGROUP * CENTROIDS, PGROUP * DSUB), jnp.bfloat16),
            pltpu.VMEM((BATCH, PGROUP * CENTROIDS), jnp.bfloat16),
        ],
    )(vecs, codebook)
